# baseline (speedup 1.0000x reference)
import functools

import jax
import jax.numpy as jnp
from jax import lax
from jax.experimental import pallas as pl
from jax.experimental.pallas import tpu as pltpu
from jax.experimental.pallas import tpu_sc as plsc

_N_VERTS = 2146689
_SH2 = 67584
_NV2P = 16 * _SH2
_NV_PAD = 2 * _NV2P
_STG = _NV_PAD // 16
_SUB = 8
_TMPW = _STG // _SUB
_PKW = _TMPW // 2
_N_EDGES = 6390784
_NC = 2
_NS = 16
_NW = _NC * _NS
_CHUNK = 2048
_CW = 2 * _CHUNK
_GROUPS = _CHUNK // 16
_NCH = 3121
_LASTOFF = 2 * _N_EDGES - _CW
_NKBASE = 97


def _sc_partials(tcat, rb):
    mesh = plsc.VectorSubcoreMesh(core_axis_name="c", subcore_axis_name="s")

    @functools.partial(
        pl.kernel,
        mesh=mesh,
        compiler_params=pltpu.CompilerParams(needs_layout_passes=False),
        out_type=jax.ShapeDtypeStruct((_NW, 16), jnp.float32),
        scratch_types=[
            pltpu.VMEM((_TMPW,), jnp.int32),
            pltpu.VMEM((_PKW,), jnp.int32),
            pltpu.VMEM((4 * _CW,), jnp.int32),
            pltpu.VMEM((3 * _CW,), jnp.int32),
            pltpu.VMEM((2 * _CW,), jnp.int32),
            pltpu.VMEM((16,), jnp.float32),
            pltpu.VMEM_SHARED((_NV2P,), jnp.int32),
            pltpu.SemaphoreType.DMA((4,)),
            pltpu.SemaphoreType.DMA((2,)),
        ],
    )
    def body(tcat_hbm, rb_hbm, out_hbm, tmp_v, pack_v, idx_v,
             hidx_v, val_v, res_v, table, sem_i, sem_g):
        cid = lax.axis_index("c")
        sid = lax.axis_index("s")
        wid = sid * _NC + cid
        n_k = _NKBASE + (wid < 17).astype(jnp.int32)
        lane = lax.broadcasted_iota(jnp.int32, (16,), 0)
        lane2 = 2 * lane
        zeros = jnp.zeros((16,), jnp.float32)
        ones = jnp.ones((16,), jnp.float32)

        for q in range(_SUB):
            pltpu.sync_copy(
                rb_hbm.at[pl.ds(sid * _STG + q * _TMPW, _TMPW)], tmp_v)

            def pk(g, _):
                base = g * 32
                we = plsc.load_gather(tmp_v, [base + lane2])
                wo = plsc.load_gather(tmp_v, [base + lane2 + 1])
                pack_v[pl.ds(g * 16, 16)] = lax.bitwise_or(
                    we, lax.shift_left(wo, 16))
                return 0

            lax.fori_loop(0, _PKW // 16, pk, 0, unroll=4)
            pltpu.sync_copy(
                pack_v, table.at[pl.ds(sid * _SH2 + q * _PKW, _PKW)])
        plsc.subcore_barrier()

        def _idx_args(k):
            b = lax.rem(k, 4)
            off = jnp.minimum((wid + _NW * k) * _CW, _LASTOFF)
            return (tcat_hbm.at[pl.ds(off, _CW)],
                    idx_v.at[pl.ds(b * _CW, _CW)], sem_i.at[b])

        def _gat_args(k):
            b = lax.rem(k, 2)
            return (table.at[hidx_v.at[pl.ds(lax.rem(k, 3) * _CW, _CW)]],
                    val_v.at[pl.ds(b * _CW, _CW)], sem_g.at[b])

        def idx_start(k):
            pltpu.async_copy(*_idx_args(k))

        def idx_wait(k):
            pltpu.make_async_copy(*_idx_args(k)).wait()

        def gather_start(k):
            pltpu.async_copy(*_gat_args(k))

        def gather_wait(k):
            pltpu.make_async_copy(*_gat_args(k)).wait()

        def mask_pass(k):
            b4 = lax.rem(k, 4) * _CW
            b3 = lax.rem(k, 3) * _CW

            def cbody(g, _):
                o = g * 16
                v = idx_v[pl.ds(b4 + o, 16)]
                hidx_v[pl.ds(b3 + o, 16)] = lax.bitwise_and(v, 0x7FFFFFFF)
                return 0

            lax.fori_loop(0, 2 * _GROUPS, cbody, 0, unroll=8)

        idx_start(0)
        idx_start(1)
        idx_start(2)
        idx_wait(0)
        mask_pass(0)
        gather_start(0)
        idx_wait(1)
        mask_pass(1)

        def chunk_body(k, carry):
            s_acc, c_acc = carry

            @pl.when(k + 1 < n_k)
            def _():
                gather_start(k + 1)

            @pl.when(k + 2 < n_k)
            def _():
                idx_wait(k + 2)
                mask_pass(k + 2)

            @pl.when(k + 3 < n_k)
            def _():
                idx_start(k + 3)

            gather_wait(k)

            b3 = lax.rem(k, 4) * _CW
            b2 = lax.rem(k, 2) * _CW
            lim = jnp.where(wid + _NW * k == _NCH - 1, _CHUNK // 2, 0)

            def group_body(j, gc):
                s, c = gc
                ao = (j // 8) * 256 + (j % 8) * 16
                bo = ao + 128
                eid = (j // 8) * 128 + (j % 8) * 16 + lane

                def fetch(o):
                    w = val_v[pl.ds(b2 + o, 16)]
                    t = idx_v[pl.ds(b3 + o, 16)]
                    odd = t < 0
                    bits = jnp.where(odd, lax.bitwise_and(w, -65536),
                                     lax.shift_left(w, 16))
                    return plsc.bitcast(bits, jnp.float32)

                a = fetch(ao)
                b = fetch(bo)
                ga = a > 0.0
                gb = b > 0.0
                mask = jnp.logical_and(ga != gb, eid >= lim)
                ua = jnp.exp(-jnp.abs(a))
                ub = jnp.exp(-jnp.abs(b))
                spa = ua * (0.9992355 + ua * (-0.49023072 + ua * (0.28527268 + ua * (-0.13158183 + ua * 0.030449))))
                spb = ub * (0.9992355 + ub * (-0.49023072 + ub * (0.28527268 + ub * (-0.13158183 + ub * 0.030449))))
                y1 = gb.astype(jnp.float32)
                y2 = ga.astype(jnp.float32)
                term = (jnp.maximum(a, 0.0) - a * y1 + spa
                        + jnp.maximum(b, 0.0) - b * y2 + spb)
                s = s + jnp.where(mask, term, zeros)
                c = c + jnp.where(mask, ones, zeros)
                return s, c

            return lax.fori_loop(0, _GROUPS, group_body, (s_acc, c_acc), unroll=4)

        s_acc, c_acc = lax.fori_loop(0, n_k, chunk_body, (zeros, zeros))
        s_tot = jnp.sum(s_acc)
        c_tot = jnp.sum(c_acc)
        res = jnp.where(lane == 0, s_tot, jnp.where(lane == 1, c_tot, 0.0))
        res_v[...] = res
        pltpu.sync_copy(res_v, out_hbm.at[wid])

    return body(tcat, rb)


def _finish(partials):
    def body(p_ref, o_ref):
        x = p_ref[...]
        s = jnp.sum(x[:, 0])
        c = jnp.sum(x[:, 1])
        o_ref[0] = s / jnp.maximum(c, 1.0)

    return pl.pallas_call(
        body,
        out_shape=jax.ShapeDtypeStruct((1,), jnp.float32),
        out_specs=pl.BlockSpec(memory_space=pltpu.SMEM),
    )(partials)


def kernel(sdf, all_edges):
    tcat = all_edges.reshape(-1, 128, 2).transpose(0, 2, 1).reshape(-1)
    tpk = lax.bitwise_or(lax.shift_right_logical(tcat, 1),
                         lax.shift_left(lax.bitwise_and(tcat, 1), 31))
    bits = lax.bitcast_convert_type(
        jnp.pad(sdf, (0, _NV_PAD - _N_VERTS)), jnp.uint32)
    rb = jnp.right_shift(
        bits + jnp.uint32(0x7FFF)
        + jnp.bitwise_and(jnp.right_shift(bits, jnp.uint32(16)),
                          jnp.uint32(1)),
        jnp.uint32(16)).astype(jnp.int32)
    partials = _sc_partials(tpk, rb)
    return _finish(partials)[0]


# speedup vs baseline: 1.0437x; 1.0437x over previous
import functools

import jax
import jax.numpy as jnp
from jax import lax
from jax.experimental import pallas as pl
from jax.experimental.pallas import tpu as pltpu
from jax.experimental.pallas import tpu_sc as plsc

_N_VERTS = 2146689
_SH2 = 67584
_NV2P = 16 * _SH2
_NV_PAD = 2 * _NV2P
_STG = _NV_PAD // 16
_SUB = 8
_TMPW = _STG // _SUB
_PKW = _TMPW // 2
_N_EDGES = 6390784
_NC = 2
_NS = 16
_NW = _NC * _NS
_CHUNK = 2048
_CW = 2 * _CHUNK
_GROUPS = _CHUNK // 16
_NCH = 3121
_LASTOFF = 2 * _N_EDGES - _CW
_NKBASE = 97


def _sc_partials(tcat, rb):
    mesh = plsc.VectorSubcoreMesh(core_axis_name="c", subcore_axis_name="s")

    @functools.partial(
        pl.kernel,
        mesh=mesh,
        compiler_params=pltpu.CompilerParams(needs_layout_passes=False),
        out_type=jax.ShapeDtypeStruct((_NW, 16), jnp.float32),
        scratch_types=[
            pltpu.VMEM((_TMPW,), jnp.int32),
            pltpu.VMEM((_PKW,), jnp.int32),
            pltpu.VMEM((3 * _CW,), jnp.int32),
            pltpu.VMEM((2 * _CW,), jnp.int32),
            pltpu.VMEM((2 * _CW,), jnp.int32),
            pltpu.VMEM((16,), jnp.float32),
            pltpu.VMEM_SHARED((_NV2P,), jnp.int32),
            pltpu.SemaphoreType.DMA((3,)),
            pltpu.SemaphoreType.DMA((2,)),
        ],
    )
    def body(tcat_hbm, rb_hbm, out_hbm, tmp_v, pack_v, idx_v,
             hidx_v, val_v, res_v, table, sem_i, sem_g):
        cid = lax.axis_index("c")
        sid = lax.axis_index("s")
        wid = sid * _NC + cid
        n_k = _NKBASE + (wid < 17).astype(jnp.int32)
        lane = lax.broadcasted_iota(jnp.int32, (16,), 0)
        lane2 = 2 * lane
        zeros = jnp.zeros((16,), jnp.float32)
        ones = jnp.ones((16,), jnp.float32)

        for q in range(_SUB):
            pltpu.sync_copy(
                rb_hbm.at[pl.ds(sid * _STG + q * _TMPW, _TMPW)], tmp_v)

            def pk(g, _):
                base = g * 32
                we = plsc.load_gather(tmp_v, [base + lane2])
                wo = plsc.load_gather(tmp_v, [base + lane2 + 1])
                pack_v[pl.ds(g * 16, 16)] = lax.bitwise_or(
                    we, lax.shift_left(wo, 16))
                return 0

            lax.fori_loop(0, _PKW // 16, pk, 0, unroll=4)
            pltpu.sync_copy(
                pack_v, table.at[pl.ds(sid * _SH2 + q * _PKW, _PKW)])
        plsc.subcore_barrier()

        def _idx_args(k):
            b = lax.rem(k, 3)
            off = jnp.minimum((wid + _NW * k) * _CW, _LASTOFF)
            return (tcat_hbm.at[pl.ds(off, _CW)],
                    idx_v.at[pl.ds(b * _CW, _CW)], sem_i.at[b])

        def _gat_args(k):
            b = lax.rem(k, 2)
            return (table.at[hidx_v.at[pl.ds(b * _CW, _CW)]],
                    val_v.at[pl.ds(b * _CW, _CW)], sem_g.at[b])

        def idx_start(k):
            pltpu.async_copy(*_idx_args(k))

        def idx_wait(k):
            pltpu.make_async_copy(*_idx_args(k)).wait()

        def gather_start(k):
            pltpu.async_copy(*_gat_args(k))

        def gather_wait(k):
            pltpu.make_async_copy(*_gat_args(k)).wait()

        def mask_pass(k):
            b3 = lax.rem(k, 3) * _CW
            b2 = lax.rem(k, 2) * _CW

            def cbody(g, _):
                o = g * 16
                v = idx_v[pl.ds(b3 + o, 16)]
                hidx_v[pl.ds(b2 + o, 16)] = lax.bitwise_and(v, 0x7FFFFFFF)
                return 0

            lax.fori_loop(0, 2 * _GROUPS, cbody, 0, unroll=8)

        idx_start(0)
        idx_wait(0)
        mask_pass(0)
        gather_start(0)
        idx_start(1)

        def chunk_body(k, carry):
            s_acc, c_acc = carry

            @pl.when(k + 1 < n_k)
            def _():
                idx_wait(k + 1)
                mask_pass(k + 1)
                gather_start(k + 1)

            gather_wait(k)

            @pl.when(k + 2 < n_k)
            def _():
                idx_start(k + 2)

            b3 = lax.rem(k, 3) * _CW
            b2 = lax.rem(k, 2) * _CW
            lim = jnp.where(wid + _NW * k == _NCH - 1, _CHUNK // 2, 0)

            def group_body(j, gc):
                s, c = gc
                ao = (j // 8) * 256 + (j % 8) * 16
                bo = ao + 128
                eid = (j // 8) * 128 + (j % 8) * 16 + lane

                def fetch(o):
                    w = val_v[pl.ds(b2 + o, 16)]
                    t = idx_v[pl.ds(b3 + o, 16)]
                    odd = t < 0
                    bits = jnp.where(odd, lax.bitwise_and(w, -65536),
                                     lax.shift_left(w, 16))
                    return plsc.bitcast(bits, jnp.float32)

                a = fetch(ao)
                b = fetch(bo)
                ga = a > 0.0
                gb = b > 0.0
                mask = jnp.logical_and(ga != gb, eid >= lim)
                ua = jnp.exp(-jnp.abs(a))
                ub = jnp.exp(-jnp.abs(b))
                spa = ua * (0.9992355 + ua * (-0.49023072 + ua * (0.28527268 + ua * (-0.13158183 + ua * 0.030449))))
                spb = ub * (0.9992355 + ub * (-0.49023072 + ub * (0.28527268 + ub * (-0.13158183 + ub * 0.030449))))
                y1 = gb.astype(jnp.float32)
                y2 = ga.astype(jnp.float32)
                term = (jnp.maximum(a, 0.0) - a * y1 + spa
                        + jnp.maximum(b, 0.0) - b * y2 + spb)
                s = s + jnp.where(mask, term, zeros)
                c = c + jnp.where(mask, ones, zeros)
                return s, c

            return lax.fori_loop(0, _GROUPS, group_body, (s_acc, c_acc), unroll=4)

        s_acc, c_acc = lax.fori_loop(0, n_k, chunk_body, (zeros, zeros))
        s_tot = jnp.sum(s_acc)
        c_tot = jnp.sum(c_acc)
        res = jnp.where(lane == 0, s_tot, jnp.where(lane == 1, c_tot, 0.0))
        res_v[...] = res
        pltpu.sync_copy(res_v, out_hbm.at[wid])

    return body(tcat, rb)


def _finish(partials):
    def body(p_ref, o_ref):
        x = p_ref[...]
        s = jnp.sum(x[:, 0])
        c = jnp.sum(x[:, 1])
        o_ref[0] = s / jnp.maximum(c, 1.0)

    return pl.pallas_call(
        body,
        out_shape=jax.ShapeDtypeStruct((1,), jnp.float32),
        out_specs=pl.BlockSpec(memory_space=pltpu.SMEM),
    )(partials)


def kernel(sdf, all_edges):
    tcat = all_edges.reshape(-1, 128, 2).transpose(0, 2, 1).reshape(-1)
    tpk = lax.bitwise_or(lax.shift_right_logical(tcat, 1),
                         lax.shift_left(lax.bitwise_and(tcat, 1), 31))
    bits = lax.bitcast_convert_type(
        jnp.pad(sdf, (0, _NV_PAD - _N_VERTS)), jnp.uint32)
    rb = jnp.right_shift(
        bits + jnp.uint32(0x7FFF)
        + jnp.bitwise_and(jnp.right_shift(bits, jnp.uint32(16)),
                          jnp.uint32(1)),
        jnp.uint32(16)).astype(jnp.int32)
    partials = _sc_partials(tpk, rb)
    return _finish(partials)[0]


# speedup vs baseline: 1.0807x; 1.0354x over previous
import functools

import jax
import jax.numpy as jnp
from jax import lax
from jax.experimental import pallas as pl
from jax.experimental.pallas import tpu as pltpu
from jax.experimental.pallas import tpu_sc as plsc

_N_VERTS = 2146689
_SH2 = 67584
_NV2P = 16 * _SH2
_NV_PAD = 2 * _NV2P
_STG = _NV_PAD // 16
_SUB = 8
_TMPW = _STG // _SUB
_PKW = _TMPW // 2
_N_EDGES = 6390784
_NC = 2
_NS = 16
_NW = _NC * _NS
_CHUNK = 2048
_CW = 2 * _CHUNK
_GROUPS = _CHUNK // 16
_NCH = 3121
_LASTOFF = 2 * _N_EDGES - _CW
_NKBASE = 97


def _sc_partials(tcat, rb):
    mesh = plsc.VectorSubcoreMesh(core_axis_name="c", subcore_axis_name="s")

    @functools.partial(
        pl.kernel,
        mesh=mesh,
        compiler_params=pltpu.CompilerParams(needs_layout_passes=False),
        out_type=jax.ShapeDtypeStruct((_NW, 16), jnp.float32),
        scratch_types=[
            pltpu.VMEM((_TMPW,), jnp.int32),
            pltpu.VMEM((_PKW,), jnp.int32),
            pltpu.VMEM((3 * _CW,), jnp.int32),
            pltpu.VMEM((2 * _CW,), jnp.int32),
            pltpu.VMEM((2 * _CW,), jnp.int32),
            pltpu.VMEM((16,), jnp.float32),
            pltpu.VMEM_SHARED((_NV2P,), jnp.int32),
            pltpu.SemaphoreType.DMA((3,)),
            pltpu.SemaphoreType.DMA((2,)),
        ],
    )
    def body(tcat_hbm, rb_hbm, out_hbm, tmp_v, pack_v, idx_v,
             hidx_v, val_v, res_v, table, sem_i, sem_g):
        cid = lax.axis_index("c")
        sid = lax.axis_index("s")
        wid = sid * _NC + cid
        n_k = _NKBASE + (wid < 17).astype(jnp.int32)
        lane = lax.broadcasted_iota(jnp.int32, (16,), 0)
        lane2 = 2 * lane
        zeros = jnp.zeros((16,), jnp.float32)
        ones = jnp.ones((16,), jnp.float32)

        for q in range(_SUB):
            pltpu.sync_copy(
                rb_hbm.at[pl.ds(sid * _STG + q * _TMPW, _TMPW)], tmp_v)

            def pk(g, _):
                base = g * 32
                we = plsc.load_gather(tmp_v, [base + lane2])
                wo = plsc.load_gather(tmp_v, [base + lane2 + 1])
                pack_v[pl.ds(g * 16, 16)] = lax.bitwise_or(
                    we, lax.shift_left(wo, 16))
                return 0

            lax.fori_loop(0, _PKW // 16, pk, 0, unroll=4)
            pltpu.sync_copy(
                pack_v, table.at[pl.ds(sid * _SH2 + q * _PKW, _PKW)])
        plsc.subcore_barrier()

        def _idx_args(k):
            b = lax.rem(k, 3)
            off = jnp.minimum((wid + _NW * k) * _CW, _LASTOFF)
            return (tcat_hbm.at[pl.ds(off, _CW)],
                    idx_v.at[pl.ds(b * _CW, _CW)], sem_i.at[b])

        def _gat_args(k):
            b = lax.rem(k, 2)
            return (table.at[hidx_v.at[pl.ds(b * _CW, _CW)]],
                    val_v.at[pl.ds(b * _CW, _CW)], sem_g.at[b])

        def idx_start(k):
            pltpu.async_copy(*_idx_args(k))

        def idx_wait(k):
            pltpu.make_async_copy(*_idx_args(k)).wait()

        def gather_start(k):
            pltpu.async_copy(*_gat_args(k))

        def gather_wait(k):
            pltpu.make_async_copy(*_gat_args(k)).wait()

        def mask_pass(k):
            b3 = lax.rem(k, 3) * _CW
            b2 = lax.rem(k, 2) * _CW

            def cbody(g, _):
                o = g * 16
                v = idx_v[pl.ds(b3 + o, 16)]
                hidx_v[pl.ds(b2 + o, 16)] = lax.bitwise_and(v, 0x7FFFFFFF)
                return 0

            lax.fori_loop(0, 2 * _GROUPS, cbody, 0, unroll=8)

        idx_start(0)
        idx_wait(0)
        mask_pass(0)
        gather_start(0)
        idx_start(1)

        def chunk_body(k, carry):
            s_acc, c_acc = carry

            @pl.when(k + 1 < n_k)
            def _():
                idx_wait(k + 1)
                mask_pass(k + 1)
                gather_start(k + 1)

            @pl.when(k + 2 < n_k)
            def _():
                idx_start(k + 2)

            gather_wait(k)

            b3 = lax.rem(k, 3) * _CW
            b2 = lax.rem(k, 2) * _CW
            lim = jnp.where(wid + _NW * k == _NCH - 1, _CHUNK // 2, 0)

            def group_body(j, gc):
                s, c = gc
                ao = (j // 8) * 256 + (j % 8) * 16
                bo = ao + 128
                eid = (j // 8) * 128 + (j % 8) * 16 + lane

                def fetch(o):
                    w = val_v[pl.ds(b2 + o, 16)]
                    t = idx_v[pl.ds(b3 + o, 16)]
                    odd = t < 0
                    bits = jnp.where(odd, lax.bitwise_and(w, -65536),
                                     lax.shift_left(w, 16))
                    return plsc.bitcast(bits, jnp.float32)

                a = fetch(ao)
                b = fetch(bo)
                mask = jnp.logical_and((a > 0.0) != (b > 0.0), eid >= lim)
                aa = jnp.abs(a)
                ab = jnp.abs(b)
                ua = jnp.exp(-aa)
                ub = jnp.exp(-ab)
                spa = ua * (0.9992355 + ua * (-0.49023072 + ua * (0.28527268 + ua * (-0.13158183 + ua * 0.030449))))
                spb = ub * (0.9992355 + ub * (-0.49023072 + ub * (0.28527268 + ub * (-0.13158183 + ub * 0.030449))))
                term = aa + ab + spa + spb
                s = s + jnp.where(mask, term, zeros)
                c = c + jnp.where(mask, ones, zeros)
                return s, c

            return lax.fori_loop(0, _GROUPS, group_body, (s_acc, c_acc), unroll=8)

        s_acc, c_acc = lax.fori_loop(0, n_k, chunk_body, (zeros, zeros))
        s_tot = jnp.sum(s_acc)
        c_tot = jnp.sum(c_acc)
        res = jnp.where(lane == 0, s_tot, jnp.where(lane == 1, c_tot, 0.0))
        res_v[...] = res
        pltpu.sync_copy(res_v, out_hbm.at[wid])

    return body(tcat, rb)


def _finish(partials):
    def body(p_ref, o_ref):
        x = p_ref[...]
        s = jnp.sum(x[:, 0])
        c = jnp.sum(x[:, 1])
        o_ref[0] = s / jnp.maximum(c, 1.0)

    return pl.pallas_call(
        body,
        out_shape=jax.ShapeDtypeStruct((1,), jnp.float32),
        out_specs=pl.BlockSpec(memory_space=pltpu.SMEM),
    )(partials)


def kernel(sdf, all_edges):
    tcat = all_edges.reshape(-1, 128, 2).transpose(0, 2, 1).reshape(-1)
    tpk = lax.bitwise_or(lax.shift_right_logical(tcat, 1),
                         lax.shift_left(lax.bitwise_and(tcat, 1), 31))
    bits = lax.bitcast_convert_type(
        jnp.pad(sdf, (0, _NV_PAD - _N_VERTS)), jnp.uint32)
    rb = jnp.right_shift(
        bits + jnp.uint32(0x7FFF)
        + jnp.bitwise_and(jnp.right_shift(bits, jnp.uint32(16)),
                          jnp.uint32(1)),
        jnp.uint32(16)).astype(jnp.int32)
    partials = _sc_partials(tpk, rb)
    return _finish(partials)[0]


# speedup vs baseline: 1.1211x; 1.0374x over previous
import functools

import jax
import jax.numpy as jnp
from jax import lax
from jax.experimental import pallas as pl
from jax.experimental.pallas import tpu as pltpu
from jax.experimental.pallas import tpu_sc as plsc

_N_VERTS = 2146689
_SH2 = 67584
_NV2P = 16 * _SH2
_NV_PAD = 2 * _NV2P
_STG = _NV_PAD // 16
_SUB = 16
_TMPW = _STG // _SUB
_PKW = _TMPW // 2
_N_EDGES = 6390784
_NC = 2
_NS = 16
_NW = _NC * _NS
_CHUNK = 2048
_CW = 2 * _CHUNK
_GROUPS = _CHUNK // 16
_NCH = 3121
_LASTOFF = 2 * _N_EDGES - _CW
_NKBASE = 97


def _sc_partials(tcat, rb):
    mesh = plsc.VectorSubcoreMesh(core_axis_name="c", subcore_axis_name="s")

    @functools.partial(
        pl.kernel,
        mesh=mesh,
        compiler_params=pltpu.CompilerParams(needs_layout_passes=False),
        out_type=jax.ShapeDtypeStruct((_NW, 16), jnp.float32),
        scratch_types=[
            pltpu.VMEM((2 * _TMPW,), jnp.int32),
            pltpu.VMEM((2 * _PKW,), jnp.int32),
            pltpu.VMEM((3 * _CW,), jnp.int32),
            pltpu.VMEM((2 * _CW,), jnp.int32),
            pltpu.VMEM((2 * _CW,), jnp.int32),
            pltpu.VMEM((16,), jnp.float32),
            pltpu.VMEM_SHARED((_NV2P,), jnp.int32),
            pltpu.SemaphoreType.DMA((3,)),
            pltpu.SemaphoreType.DMA((2,)),
            pltpu.SemaphoreType.DMA((2,)),
            pltpu.SemaphoreType.DMA((2,)),
        ],
    )
    def body(tcat_hbm, rb_hbm, out_hbm, tmp_v, pack_v, idx_v,
             hidx_v, val_v, res_v, table, sem_i, sem_g, sem_si, sem_so):
        cid = lax.axis_index("c")
        sid = lax.axis_index("s")
        wid = sid * _NC + cid
        n_k = _NKBASE + (wid < 17).astype(jnp.int32)
        lane = lax.broadcasted_iota(jnp.int32, (16,), 0)
        lane2 = 2 * lane
        zeros = jnp.zeros((16,), jnp.float32)
        ones = jnp.ones((16,), jnp.float32)

        def _sin_args(q):
            return (rb_hbm.at[pl.ds(sid * _STG + q * _TMPW, _TMPW)],
                    tmp_v.at[pl.ds((q % 2) * _TMPW, _TMPW)],
                    sem_si.at[q % 2])

        def _sout_args(q):
            return (pack_v.at[pl.ds((q % 2) * _PKW, _PKW)],
                    table.at[pl.ds(sid * _SH2 + q * _PKW, _PKW)],
                    sem_so.at[q % 2])

        pltpu.async_copy(*_sin_args(0))
        for q in range(_SUB):
            pltpu.make_async_copy(*_sin_args(q)).wait()
            if q + 1 < _SUB:
                pltpu.async_copy(*_sin_args(q + 1))
            if q >= 2:
                pltpu.make_async_copy(*_sout_args(q - 2)).wait()
            tb = (q % 2) * _TMPW
            pb = (q % 2) * _PKW

            def pk(g, _):
                base = tb + g * 32
                we = plsc.load_gather(tmp_v, [base + lane2])
                wo = plsc.load_gather(tmp_v, [base + lane2 + 1])
                pack_v[pl.ds(pb + g * 16, 16)] = lax.bitwise_or(
                    we, lax.shift_left(wo, 16))
                return 0

            lax.fori_loop(0, _PKW // 16, pk, 0, unroll=4)
            pltpu.async_copy(*_sout_args(q))
        pltpu.make_async_copy(*_sout_args(_SUB - 2)).wait()
        pltpu.make_async_copy(*_sout_args(_SUB - 1)).wait()
        plsc.subcore_barrier()

        def _idx_args(k):
            b = lax.rem(k, 3)
            off = jnp.minimum((wid + _NW * k) * _CW, _LASTOFF)
            return (tcat_hbm.at[pl.ds(off, _CW)],
                    idx_v.at[pl.ds(b * _CW, _CW)], sem_i.at[b])

        def _gat_args(k):
            b = lax.rem(k, 2)
            return (table.at[hidx_v.at[pl.ds(b * _CW, _CW)]],
                    val_v.at[pl.ds(b * _CW, _CW)], sem_g.at[b])

        def idx_start(k):
            pltpu.async_copy(*_idx_args(k))

        def idx_wait(k):
            pltpu.make_async_copy(*_idx_args(k)).wait()

        def gather_start(k):
            pltpu.async_copy(*_gat_args(k))

        def gather_wait(k):
            pltpu.make_async_copy(*_gat_args(k)).wait()

        def mask_pass(k):
            b3 = lax.rem(k, 3) * _CW
            b2 = lax.rem(k, 2) * _CW

            def cbody(g, _):
                o = g * 16
                v = idx_v[pl.ds(b3 + o, 16)]
                hidx_v[pl.ds(b2 + o, 16)] = lax.bitwise_and(v, 0x7FFFFFFF)
                return 0

            lax.fori_loop(0, 2 * _GROUPS, cbody, 0, unroll=8)

        idx_start(0)
        idx_wait(0)
        mask_pass(0)
        gather_start(0)
        idx_start(1)

        def chunk_body(k, carry):
            s_acc, c_acc = carry

            @pl.when(k + 1 < n_k)
            def _():
                idx_wait(k + 1)
                mask_pass(k + 1)
                gather_start(k + 1)

            @pl.when(k + 2 < n_k)
            def _():
                idx_start(k + 2)

            gather_wait(k)

            b3 = lax.rem(k, 3) * _CW
            b2 = lax.rem(k, 2) * _CW
            lim = jnp.where(wid + _NW * k == _NCH - 1, _CHUNK // 2, 0)

            def group_body(j, gc):
                s, c = gc
                ao = (j // 8) * 256 + (j % 8) * 16
                bo = ao + 128
                eid = (j // 8) * 128 + (j % 8) * 16 + lane

                def fetch(o):
                    w = val_v[pl.ds(b2 + o, 16)]
                    t = idx_v[pl.ds(b3 + o, 16)]
                    odd = t < 0
                    bits = jnp.where(odd, lax.bitwise_and(w, -65536),
                                     lax.shift_left(w, 16))
                    return plsc.bitcast(bits, jnp.float32)

                a = fetch(ao)
                b = fetch(bo)
                mask = jnp.logical_and((a > 0.0) != (b > 0.0), eid >= lim)
                aa = jnp.abs(a)
                ab = jnp.abs(b)
                ua = jnp.exp(-aa)
                ub = jnp.exp(-ab)
                spa = ua * (0.9992355 + ua * (-0.49023072 + ua * (0.28527268 + ua * (-0.13158183 + ua * 0.030449))))
                spb = ub * (0.9992355 + ub * (-0.49023072 + ub * (0.28527268 + ub * (-0.13158183 + ub * 0.030449))))
                term = aa + ab + spa + spb
                s = s + jnp.where(mask, term, zeros)
                c = c + jnp.where(mask, ones, zeros)
                return s, c

            return lax.fori_loop(0, _GROUPS, group_body, (s_acc, c_acc), unroll=8)

        s_acc, c_acc = lax.fori_loop(0, n_k, chunk_body, (zeros, zeros))
        s_tot = jnp.sum(s_acc)
        c_tot = jnp.sum(c_acc)
        res = jnp.where(lane == 0, s_tot, jnp.where(lane == 1, c_tot, 0.0))
        res_v[...] = res
        pltpu.sync_copy(res_v, out_hbm.at[wid])

    return body(tcat, rb)


def _finish(partials):
    def body(p_ref, o_ref):
        x = p_ref[...]
        s = jnp.sum(x[:, 0])
        c = jnp.sum(x[:, 1])
        o_ref[0] = s / jnp.maximum(c, 1.0)

    return pl.pallas_call(
        body,
        out_shape=jax.ShapeDtypeStruct((1,), jnp.float32),
        out_specs=pl.BlockSpec(memory_space=pltpu.SMEM),
    )(partials)


def kernel(sdf, all_edges):
    tcat = all_edges.reshape(-1, 128, 2).transpose(0, 2, 1).reshape(-1)
    tpk = lax.bitwise_or(lax.shift_right_logical(tcat, 1),
                         lax.shift_left(lax.bitwise_and(tcat, 1), 31))
    bits = lax.bitcast_convert_type(
        jnp.pad(sdf, (0, _NV_PAD - _N_VERTS)), jnp.uint32)
    rb = jnp.right_shift(
        bits + jnp.uint32(0x7FFF)
        + jnp.bitwise_and(jnp.right_shift(bits, jnp.uint32(16)),
                          jnp.uint32(1)),
        jnp.uint32(16)).astype(jnp.int32)
    partials = _sc_partials(tpk, rb)
    return _finish(partials)[0]


# speedup vs baseline: 1.2467x; 1.1120x over previous
import functools

import jax
import jax.numpy as jnp
from jax import lax
from jax.experimental import pallas as pl
from jax.experimental.pallas import tpu as pltpu
from jax.experimental.pallas import tpu_sc as plsc

_N_VERTS = 2146689
_SH2 = 67584
_NV2P = 16 * _SH2
_NV_PAD = 2 * _NV2P
_STG = _NV_PAD // 16
_SUB = 16
_TMPW = _STG // _SUB
_PKW = _TMPW // 2
_N_EDGES = 6390784
_NC = 2
_NS = 16
_NW = _NC * _NS
_CHUNK = 2048
_CW = 2 * _CHUNK
_GROUPS = _CHUNK // 16
_NCH = 3121
_LASTOFF = 2 * _N_EDGES - _CW
_NKBASE = 97


def _sc_partials(tcat, rb):
    mesh = plsc.VectorSubcoreMesh(core_axis_name="c", subcore_axis_name="s")

    @functools.partial(
        pl.kernel,
        mesh=mesh,
        compiler_params=pltpu.CompilerParams(needs_layout_passes=False),
        out_type=jax.ShapeDtypeStruct((_NW, 16), jnp.float32),
        scratch_types=[
            pltpu.VMEM((2 * _TMPW,), jnp.int32),
            pltpu.VMEM((2 * _PKW,), jnp.int32),
            pltpu.VMEM((3 * _CW,), jnp.int32),
            pltpu.VMEM((2 * _CW,), jnp.int32),
            pltpu.VMEM((2 * _CW,), jnp.int32),
            pltpu.VMEM((16,), jnp.float32),
            pltpu.VMEM_SHARED((_NV2P,), jnp.int32),
            pltpu.SemaphoreType.DMA((3,)),
            pltpu.SemaphoreType.DMA((2,)),
            pltpu.SemaphoreType.DMA((2,)),
            pltpu.SemaphoreType.DMA((2,)),
        ],
    )
    def body(tcat_hbm, rb_hbm, out_hbm, tmp_v, pack_v, idx_v,
             hidx_v, val_v, res_v, table, sem_i, sem_g, sem_si, sem_so):
        cid = lax.axis_index("c")
        sid = lax.axis_index("s")
        wid = sid * _NC + cid
        n_k = _NKBASE + (wid < 17).astype(jnp.int32)
        lane = lax.broadcasted_iota(jnp.int32, (16,), 0)
        lane2 = 2 * lane
        zeros = jnp.zeros((16,), jnp.float32)
        ones = jnp.ones((16,), jnp.float32)

        def _sin_args(q):
            return (rb_hbm.at[pl.ds(sid * _STG + q * _TMPW, _TMPW)],
                    tmp_v.at[pl.ds((q % 2) * _TMPW, _TMPW)],
                    sem_si.at[q % 2])

        def _sout_args(q):
            return (pack_v.at[pl.ds((q % 2) * _PKW, _PKW)],
                    table.at[pl.ds(sid * _SH2 + q * _PKW, _PKW)],
                    sem_so.at[q % 2])

        pltpu.async_copy(*_sin_args(0))
        for q in range(_SUB):
            pltpu.make_async_copy(*_sin_args(q)).wait()
            if q + 1 < _SUB:
                pltpu.async_copy(*_sin_args(q + 1))
            if q >= 2:
                pltpu.make_async_copy(*_sout_args(q - 2)).wait()
            tb = (q % 2) * _TMPW
            pb = (q % 2) * _PKW

            def pk(g, _):
                base = tb + g * 32
                we = plsc.load_gather(tmp_v, [base + lane2])
                wo = plsc.load_gather(tmp_v, [base + lane2 + 1])
                pack_v[pl.ds(pb + g * 16, 16)] = lax.bitwise_or(
                    we, lax.shift_left(wo, 16))
                return 0

            lax.fori_loop(0, _PKW // 16, pk, 0, unroll=4)
            pltpu.async_copy(*_sout_args(q))
        pltpu.make_async_copy(*_sout_args(_SUB - 2)).wait()
        pltpu.make_async_copy(*_sout_args(_SUB - 1)).wait()
        plsc.subcore_barrier()

        def _idx_args(k):
            b = lax.rem(k, 3)
            off = jnp.minimum((wid + _NW * k) * _CW, _LASTOFF)
            return (tcat_hbm.at[pl.ds(off, _CW)],
                    idx_v.at[pl.ds(b * _CW, _CW)], sem_i.at[b])

        def _gat_args(k):
            b = lax.rem(k, 2)
            return (table.at[hidx_v.at[pl.ds(b * _CW, _CW)]],
                    val_v.at[pl.ds(b * _CW, _CW)], sem_g.at[b])

        def idx_start(k):
            pltpu.async_copy(*_idx_args(k))

        def idx_wait(k):
            pltpu.make_async_copy(*_idx_args(k)).wait()

        def gather_start(k):
            pltpu.async_copy(*_gat_args(k))

        def gather_wait(k):
            pltpu.make_async_copy(*_gat_args(k)).wait()

        def mask_pass(k):
            b3 = lax.rem(k, 3) * _CW
            b2 = lax.rem(k, 2) * _CW

            def cbody(g, _):
                o = g * 16
                v = idx_v[pl.ds(b3 + o, 16)]
                hidx_v[pl.ds(b2 + o, 16)] = lax.shift_right_logical(v, 1)
                return 0

            lax.fori_loop(0, 2 * _GROUPS, cbody, 0, unroll=8)

        idx_start(0)
        idx_wait(0)
        mask_pass(0)
        gather_start(0)
        idx_start(1)

        def chunk_body(k, carry):
            s_acc, c_acc = carry

            @pl.when(k + 1 < n_k)
            def _():
                idx_wait(k + 1)
                mask_pass(k + 1)
                gather_start(k + 1)

            @pl.when(k + 2 < n_k)
            def _():
                idx_start(k + 2)

            gather_wait(k)

            b3 = lax.rem(k, 3) * _CW
            b2 = lax.rem(k, 2) * _CW
            lim = jnp.where(wid + _NW * k == _NCH - 1, _CHUNK // 2, 0)

            def group_body(j, gc):
                s, c = gc
                ao = (j // 8) * 256 + (j % 8) * 16
                bo = ao + 128
                eid = (j // 8) * 128 + (j % 8) * 16 + lane

                def fetch(o):
                    w = val_v[pl.ds(b2 + o, 16)]
                    t = idx_v[pl.ds(b3 + o, 16)]
                    odd = lax.bitwise_and(t, 1) == 1
                    bits = jnp.where(odd, lax.bitwise_and(w, -65536),
                                     lax.shift_left(w, 16))
                    return plsc.bitcast(bits, jnp.float32)

                a = fetch(ao)
                b = fetch(bo)
                mask = jnp.logical_and((a > 0.0) != (b > 0.0), eid >= lim)
                aa = jnp.abs(a)
                ab = jnp.abs(b)
                ua = jnp.exp(-aa)
                ub = jnp.exp(-ab)
                spa = ua * (0.9992355 + ua * (-0.49023072 + ua * (0.28527268 + ua * (-0.13158183 + ua * 0.030449))))
                spb = ub * (0.9992355 + ub * (-0.49023072 + ub * (0.28527268 + ub * (-0.13158183 + ub * 0.030449))))
                term = aa + ab + spa + spb
                s = s + jnp.where(mask, term, zeros)
                c = c + jnp.where(mask, ones, zeros)
                return s, c

            return lax.fori_loop(0, _GROUPS, group_body, (s_acc, c_acc), unroll=8)

        s_acc, c_acc = lax.fori_loop(0, n_k, chunk_body, (zeros, zeros))
        s_tot = jnp.sum(s_acc)
        c_tot = jnp.sum(c_acc)
        res = jnp.where(lane == 0, s_tot, jnp.where(lane == 1, c_tot, 0.0))
        res_v[...] = res
        pltpu.sync_copy(res_v, out_hbm.at[wid])

    return body(tcat, rb)


def _finish(partials):
    def body(p_ref, o_ref):
        x = p_ref[...]
        s = jnp.sum(x[:, 0])
        c = jnp.sum(x[:, 1])
        o_ref[0] = s / jnp.maximum(c, 1.0)

    return pl.pallas_call(
        body,
        out_shape=jax.ShapeDtypeStruct((1,), jnp.float32),
        out_specs=pl.BlockSpec(memory_space=pltpu.SMEM),
    )(partials)


def kernel(sdf, all_edges):
    tpk = all_edges.reshape(-1, 128, 2).transpose(0, 2, 1).reshape(-1)
    bits = lax.bitcast_convert_type(
        jnp.pad(sdf, (0, _NV_PAD - _N_VERTS)), jnp.uint32)
    rb = jnp.right_shift(
        bits + jnp.uint32(0x7FFF)
        + jnp.bitwise_and(jnp.right_shift(bits, jnp.uint32(16)),
                          jnp.uint32(1)),
        jnp.uint32(16)).astype(jnp.int32)
    partials = _sc_partials(tpk, rb)
    return _finish(partials)[0]
